# Initial kernel scaffold; baseline (speedup 1.0000x reference)
import jax
import jax.numpy as jnp
from jax.experimental import pallas as pl


def kernel(data, W, edges_pos, edges_neg):
    raise NotImplementedError("write your pallas kernel here")


# speedup vs baseline: 5.9653x; 5.9653x over previous
import functools

import jax
import jax.numpy as jnp
from jax import lax
from jax.experimental import pallas as pl
from jax.experimental.pallas import tpu as pltpu
from jax.experimental.pallas import tpu_sc as plsc

N_NODES_ = 10000
D_ = 128
K_ = 64
E_PER = 320000
E_TOT = 2 * E_PER
NC_, NS_, LANES_ = 2, 16, 16
NW_ = NC_ * NS_
CHUNK_ = 128
NCHUNK_ = E_TOT // CHUNK_


def _mm_body(x_ref, w_ref, o_ref):
    o_ref[...] = jnp.dot(x_ref[...], w_ref[...],
                         preferred_element_type=jnp.float32)


def _encode(data, W):
    return pl.pallas_call(
        _mm_body,
        out_shape=jax.ShapeDtypeStruct((N_NODES_, K_), jnp.float32),
        grid=(5,),
        in_specs=[
            pl.BlockSpec((N_NODES_ // 5, D_), lambda i: (i, 0)),
            pl.BlockSpec((D_, K_), lambda i: (0, 0)),
        ],
        out_specs=pl.BlockSpec((N_NODES_ // 5, K_), lambda i: (i, 0)),
    )(data, W)


def _sc_scores(z, srcs, dsts):
    mesh = plsc.VectorSubcoreMesh(core_axis_name="c", subcore_axis_name="s")

    @functools.partial(
        pl.kernel,
        mesh=mesh,
        compiler_params=pltpu.CompilerParams(
            needs_layout_passes=False, use_tc_tiling_on_sc=False),
        out_type=jax.ShapeDtypeStruct((E_TOT,), jnp.float32),
        scratch_types=[
            pltpu.VMEM((CHUNK_,), jnp.int32),
            pltpu.VMEM((CHUNK_,), jnp.int32),
            pltpu.VMEM((CHUNK_, K_), jnp.float32),
            pltpu.VMEM((CHUNK_, K_), jnp.float32),
            pltpu.VMEM((CHUNK_,), jnp.float32),
            pltpu.SemaphoreType.DMA,
        ],
    )
    def k(z_hbm, src_hbm, dst_hbm, out_hbm,
          idx_s, idx_d, rows_s, rows_d, score_v, sem):
        wid = lax.axis_index("s") * NC_ + lax.axis_index("c")
        nch = jnp.where(wid < (NCHUNK_ % NW_), NCHUNK_ // NW_ + 1,
                        NCHUNK_ // NW_)

        def chunk_body(c, carry):
            off = (c * NW_ + wid) * CHUNK_
            pltpu.sync_copy(src_hbm.at[pl.ds(off, CHUNK_)], idx_s)
            pltpu.sync_copy(dst_hbm.at[pl.ds(off, CHUNK_)], idx_d)
            cp1 = pltpu.async_copy(z_hbm.at[idx_s], rows_s, sem)
            cp2 = pltpu.async_copy(z_hbm.at[idx_d], rows_d, sem)
            cp1.wait()
            cp2.wait()

            def group(g, carry2):
                base = g * LANES_
                lane = lax.iota(jnp.int32, LANES_)
                res = jnp.zeros((LANES_,), jnp.float32)
                for j in range(LANES_):
                    e = base + j
                    acc = rows_s[e, pl.ds(0, LANES_)] * rows_d[e, pl.ds(0, LANES_)]
                    for c in range(1, K_ // LANES_):
                        acc = acc + (rows_s[e, pl.ds(c * LANES_, LANES_)]
                                     * rows_d[e, pl.ds(c * LANES_, LANES_)])
                    s = jnp.sum(acc)
                    res = jnp.where(lane == j, s, res)
                score_v[pl.ds(base, LANES_)] = res
                return carry2

            lax.fori_loop(0, CHUNK_ // LANES_, group, 0)
            pltpu.sync_copy(score_v, out_hbm.at[pl.ds(off, CHUNK_)])
            return carry

        lax.fori_loop(0, nch, chunk_body, 0)

    return k(z, srcs, dsts)


def _bce_body(x_ref, o_ref):
    x = x_ref[...]
    rows = lax.broadcasted_iota(jnp.int32, x.shape, 0)
    t = (rows < (E_PER // x.shape[1])).astype(jnp.float32)
    term = jnp.maximum(x, 0.0) - x * t + jnp.log1p(jnp.exp(-jnp.abs(x)))
    o_ref[...] = (jnp.sum(term) * (1.0 / E_TOT)).reshape(1, 1)


def _bce_reduce(scores2d):
    return pl.pallas_call(
        _bce_body,
        out_shape=jax.ShapeDtypeStruct((1, 1), jnp.float32),
    )(scores2d)


def kernel(data, W, edges_pos, edges_neg):
    z = _encode(data, W)
    srcs = jnp.concatenate(
        (edges_pos[0], edges_neg[0])).astype(jnp.int32)
    dsts = jnp.concatenate(
        (edges_pos[1], edges_neg[1])).astype(jnp.int32)
    scores = _sc_scores(z, srcs, dsts)
    cost = _bce_reduce(scores.reshape(E_TOT // D_, D_))
    return cost.reshape(())
